# Initial kernel scaffold; baseline (speedup 1.0000x reference)
import jax
import jax.numpy as jnp
from jax.experimental import pallas as pl


def kernel(x, edge_index, edge_label_index, W1, b1, W2, b2, PW1, PB1, PW2, PB2, PW3, PB3):
    raise NotImplementedError("write your pallas kernel here")


# speedup vs baseline: 8.9723x; 8.9723x over previous
import functools

import jax
import jax.numpy as jnp
from jax import lax
from jax.experimental import pallas as pl
from jax.experimental.pallas import tpu as pltpu
from jax.experimental.pallas import tpu_sc as plsc

N = 10000
NP = 10240
D = 256
DH = 128
E = 160000
EP = 163840
P = 65536
CH = 128
NC = 2
NS = 16
RT = NP // NS
ECH = EP // CH
ET = ECH // NS
DCH = ECH // NC // NS
PCH = P // CH
PT = PCH // NS
BR = 1024
BP = 2048

_MESH = plsc.VectorSubcoreMesh(core_axis_name="c", subcore_axis_name="s",
                               num_cores=NC, num_subcores=NS)


@functools.partial(
    pl.kernel,
    out_type=jax.ShapeDtypeStruct((NC * NP,), jnp.float32),
    mesh=_MESH,
    scratch_types=[
        pltpu.VMEM((CH,), jnp.int32),
        pltpu.VMEM((CH,), jnp.float32),
        pltpu.VMEM((RT,), jnp.float32),
        pltpu.VMEM_SHARED((NP,), jnp.float32),
    ],
)
def _deg_kernel(dstc_hbm, out_hbm, idx_v, ones_v, zeros_v, hist_sh):
    c = lax.axis_index("c")
    s = lax.axis_index("s")
    for j in range(CH // 16):
        ones_v[pl.ds(j * 16, 16)] = jnp.full((16,), 1.0, jnp.float32)
    for j in range(RT // 16):
        zeros_v[pl.ds(j * 16, 16)] = jnp.zeros((16,), jnp.float32)
    pltpu.sync_copy(zeros_v, hist_sh.at[pl.ds(s * RT, RT)])
    plsc.subcore_barrier()
    base = (c * NS + s) * DCH

    def body(i, carry):
        pltpu.sync_copy(dstc_hbm.at[base + i], idx_v)
        pltpu.sync_copy(ones_v, hist_sh.at[idx_v], add=True)
        return carry

    lax.fori_loop(0, DCH, body, 0)
    plsc.subcore_barrier()
    out_off = pl.multiple_of(c * NP + s * RT, RT)
    pltpu.sync_copy(hist_sh.at[pl.ds(s * RT, RT)], out_hbm.at[pl.ds(out_off, RT)])


@functools.partial(
    pl.kernel,
    out_type=jax.ShapeDtypeStruct((NC * NP, DH), jnp.float32),
    mesh=_MESH,
    scratch_types=[
        pltpu.VMEM((2, CH), jnp.int32),
        pltpu.VMEM((CH, DH), jnp.float32),
        pltpu.VMEM_SHARED((NP, DH), jnp.float32),
        pltpu.SemaphoreType.DMA,
    ],
)
def _mp_kernel(h_hbm, sd_hbm, acc_hbm, sd_v, rows_v, acc_sh, sem):
    c = lax.axis_index("c")
    s = lax.axis_index("s")
    r0 = s * RT
    h_off = pl.multiple_of(c * NP + r0, RT)
    pltpu.sync_copy(h_hbm.at[pl.ds(h_off, RT)], acc_sh.at[pl.ds(r0, RT)])
    plsc.subcore_barrier()
    base = c * ECH + s * ET

    def body(i, carry):
        pltpu.sync_copy(sd_hbm.at[base + i], sd_v)
        pltpu.async_copy(h_hbm.at[sd_v.at[0]], rows_v, sem).wait()
        pltpu.sync_copy(rows_v, acc_sh.at[sd_v.at[1]], add=True)
        return carry

    lax.fori_loop(0, ET, body, 0)
    plsc.subcore_barrier()
    pltpu.sync_copy(acc_sh.at[pl.ds(r0, RT)], acc_hbm.at[pl.ds(h_off, RT)])


@functools.partial(
    pl.kernel,
    out_type=jax.ShapeDtypeStruct((NC * P, DH), jnp.float32),
    mesh=_MESH,
    scratch_types=[
        pltpu.VMEM((2, CH), jnp.int32),
        pltpu.VMEM((CH, DH), jnp.float32),
        pltpu.VMEM((CH, DH), jnp.float32),
        pltpu.SemaphoreType.DMA,
        pltpu.SemaphoreType.DMA,
    ],
)
def _dec_kernel(u_hbm, v_hbm, rc_hbm, g_hbm, rc_v, urows_v, vrows_v, sem1, sem2):
    c = lax.axis_index("c")
    s = lax.axis_index("s")
    base = c * PCH + s * PT

    def body(i, carry):
        pltpu.sync_copy(rc_hbm.at[base + i], rc_v)
        du = pltpu.async_copy(u_hbm.at[rc_v.at[0]], urows_v, sem1)
        dv = pltpu.async_copy(v_hbm.at[rc_v.at[1]], vrows_v, sem2)
        du.wait()
        dv.wait()

        def addrow(r, cr):
            for j in range(DH // 16):
                sl = pl.ds(j * 16, 16)
                urows_v[r, sl] = urows_v[r, sl] + vrows_v[r, sl]
            return cr

        lax.fori_loop(0, CH, addrow, 0)
        out_off = pl.multiple_of(c * P + (s * PT + i) * CH, CH)
        pltpu.sync_copy(urows_v, g_hbm.at[pl.ds(out_off, CH)])
        return carry

    lax.fori_loop(0, PT, body, 0)


def _tc1_body(x_ref, w_ref, degs_ref, h_ref, dinv_ref):
    deg = degs_ref[0] + degs_ref[1] + 1.0
    dinv = lax.rsqrt(deg)
    h = jnp.dot(x_ref[...], w_ref[...], preferred_element_type=jnp.float32) * dinv
    h_ref[0] = h[:, :DH]
    h_ref[1] = h[:, DH:]
    dinv_ref[...] = dinv


def _tc1(x_pad, w1, degs3):
    return pl.pallas_call(
        _tc1_body,
        grid=(NP // BR,),
        in_specs=[
            pl.BlockSpec((BR, D), lambda i: (i, 0)),
            pl.BlockSpec((D, D), lambda i: (0, 0)),
            pl.BlockSpec((NC, BR, 1), lambda i: (0, i, 0)),
        ],
        out_specs=[
            pl.BlockSpec((NC, BR, DH), lambda i: (0, i, 0)),
            pl.BlockSpec((BR, 1), lambda i: (i, 0)),
        ],
        out_shape=[
            jax.ShapeDtypeStruct((NC, NP, DH), jnp.float32),
            jax.ShapeDtypeStruct((NP, 1), jnp.float32),
        ],
    )(x_pad, w1, degs3)


def _tc2_body(acc_ref, dinv_ref, b_ref, w_ref, out_ref):
    accf = jnp.concatenate([acc_ref[0], acc_ref[1]], axis=-1)
    z = jnp.maximum(accf * dinv_ref[...] + b_ref[...], 0.0)
    h = jnp.dot(z, w_ref[...], preferred_element_type=jnp.float32) * dinv_ref[...]
    out_ref[0] = h[:, :DH]
    out_ref[1] = h[:, DH:]


def _tc2(acc3, dinv, b, w):
    return pl.pallas_call(
        _tc2_body,
        grid=(NP // BR,),
        in_specs=[
            pl.BlockSpec((NC, BR, DH), lambda i: (0, i, 0)),
            pl.BlockSpec((BR, 1), lambda i: (i, 0)),
            pl.BlockSpec((1, D), lambda i: (0, 0)),
            pl.BlockSpec((D, D), lambda i: (0, 0)),
        ],
        out_specs=pl.BlockSpec((NC, BR, DH), lambda i: (0, i, 0)),
        out_shape=jax.ShapeDtypeStruct((NC, NP, DH), jnp.float32),
    )(acc3, dinv, b, w)


def _tc3_body(acc_ref, dinv_ref, b_ref, pw1a_ref, pw1b_ref, pb1_ref, u_ref, v_ref):
    accf = jnp.concatenate([acc_ref[0], acc_ref[1]], axis=-1)
    z = jnp.maximum(accf * dinv_ref[...] + b_ref[...], 0.0)
    u = jnp.dot(z, pw1a_ref[...], preferred_element_type=jnp.float32) + pb1_ref[...]
    v = jnp.dot(z, pw1b_ref[...], preferred_element_type=jnp.float32)
    u_ref[0] = u[:, :DH]
    u_ref[1] = u[:, DH:]
    v_ref[0] = v[:, :DH]
    v_ref[1] = v[:, DH:]


def _tc3(acc3, dinv, b, pw1a, pw1b, pb1):
    return pl.pallas_call(
        _tc3_body,
        grid=(NP // BR,),
        in_specs=[
            pl.BlockSpec((NC, BR, DH), lambda i: (0, i, 0)),
            pl.BlockSpec((BR, 1), lambda i: (i, 0)),
            pl.BlockSpec((1, D), lambda i: (0, 0)),
            pl.BlockSpec((D, D), lambda i: (0, 0)),
            pl.BlockSpec((D, D), lambda i: (0, 0)),
            pl.BlockSpec((1, D), lambda i: (0, 0)),
        ],
        out_specs=[
            pl.BlockSpec((NC, BR, DH), lambda i: (0, i, 0)),
            pl.BlockSpec((NC, BR, DH), lambda i: (0, i, 0)),
        ],
        out_shape=[
            jax.ShapeDtypeStruct((NC, NP, DH), jnp.float32),
            jax.ShapeDtypeStruct((NC, NP, DH), jnp.float32),
        ],
    )(acc3, dinv, b, pw1a, pw1b, pb1)


def _tc4_body(g_ref, pw2_ref, pb2_ref, pw3_ref, pb3_ref, out_ref):
    g = jnp.concatenate([g_ref[0], g_ref[1]], axis=-1)
    h1 = jnp.maximum(g, 0.0)
    h2 = jnp.maximum(
        jnp.dot(h1, pw2_ref[...], preferred_element_type=jnp.float32) + pb2_ref[...],
        0.0)
    o = jnp.dot(h2, pw3_ref[...], preferred_element_type=jnp.float32) + pb3_ref[0, 0]
    out_ref[...] = o.reshape(BP // 128, 128)


def _tc4(g3, pw2, pb2, pw3, pb3):
    return pl.pallas_call(
        _tc4_body,
        grid=(P // BP,),
        in_specs=[
            pl.BlockSpec((NC, BP, DH), lambda i: (0, i, 0)),
            pl.BlockSpec((D, DH), lambda i: (0, 0)),
            pl.BlockSpec((1, DH), lambda i: (0, 0)),
            pl.BlockSpec((DH, 1), lambda i: (0, 0)),
            pl.BlockSpec((1, 1), lambda i: (0, 0)),
        ],
        out_specs=pl.BlockSpec((BP // 128, 128), lambda i: (i, 0)),
        out_shape=jax.ShapeDtypeStruct((P // 128, 128), jnp.float32),
    )(g3, pw2, pb2, pw3, pb3)


def kernel(x, edge_index, edge_label_index, W1, b1, W2, b2,
           PW1, PB1, PW2, PB2, PW3, PB3):
    src = edge_index[0].astype(jnp.int32)
    dst = edge_index[1].astype(jnp.int32)
    padidx = N + (jnp.arange(EP - E, dtype=jnp.int32) % (NP - N))
    srcp = jnp.concatenate([src, padidx])
    dstp = jnp.concatenate([dst, padidx])
    s_c = srcp.reshape(ECH, CH)
    d_c = dstp.reshape(ECH, CH)
    sd = jnp.concatenate([
        jnp.stack([s_c, d_c], axis=1),
        jnp.stack([s_c + NP, d_c], axis=1),
    ], axis=0)

    row = edge_label_index[0].astype(jnp.int32).reshape(PCH, CH)
    col = edge_label_index[1].astype(jnp.int32).reshape(PCH, CH)
    rc = jnp.concatenate([
        jnp.stack([row, col], axis=1),
        jnp.stack([row + NP, col + NP], axis=1),
    ], axis=0)

    x_pad = jnp.pad(x, ((0, NP - N), (0, 0)))

    degs = _deg_kernel(d_c)
    h1f, dinv = _tc1(x_pad, W1, degs.reshape(NC, NP, 1))
    acc1 = _mp_kernel(h1f.reshape(NC * NP, DH), sd)
    h2f = _tc2(acc1.reshape(NC, NP, DH), dinv, b1.reshape(1, D), W2)
    acc2 = _mp_kernel(h2f.reshape(NC * NP, DH), sd)
    uf, vf = _tc3(acc2.reshape(NC, NP, DH), dinv, b2.reshape(1, D),
                  PW1[:D], PW1[D:], PB1.reshape(1, D))
    g = _dec_kernel(uf.reshape(NC * NP, DH), vf.reshape(NC * NP, DH), rc)
    out = _tc4(g.reshape(NC, P, DH), PW2, PB2.reshape(1, DH),
               PW3, PB3.reshape(1, 1))
    return out.reshape(-1)


# speedup vs baseline: 11.3408x; 1.2640x over previous
import functools

import jax
import jax.numpy as jnp
from jax import lax
from jax.experimental import pallas as pl
from jax.experimental.pallas import tpu as pltpu
from jax.experimental.pallas import tpu_sc as plsc

N = 10000
NP = 10240
D = 256
DH = 128
E = 160000
EP = 163840
P = 65536
CH = 128
NC = 2
NS = 16
RT = NP // NS
ECH = EP // CH
ET = ECH // NS
DCH = ECH // NC // NS
PCH = P // CH
PT = PCH // NS
BR = 1024
BP = 2048

_MESH = plsc.VectorSubcoreMesh(core_axis_name="c", subcore_axis_name="s",
                               num_cores=NC, num_subcores=NS)


@functools.partial(
    pl.kernel,
    out_type=jax.ShapeDtypeStruct((NC * NP,), jnp.float32),
    mesh=_MESH,
    scratch_types=[
        pltpu.VMEM((DCH, CH), jnp.int32),
        pltpu.VMEM((CH,), jnp.float32),
        pltpu.VMEM((RT,), jnp.float32),
        pltpu.VMEM_SHARED((NP,), jnp.float32),
        pltpu.SemaphoreType.DMA,
    ],
)
def _deg_kernel(dstc_hbm, out_hbm, idx_v, ones_v, zeros_v, hist_sh, sd):
    c = lax.axis_index("c")
    s = lax.axis_index("s")
    for j in range(CH // 16):
        ones_v[pl.ds(j * 16, 16)] = jnp.full((16,), 1.0, jnp.float32)
    for j in range(RT // 16):
        zeros_v[pl.ds(j * 16, 16)] = jnp.zeros((16,), jnp.float32)
    pltpu.sync_copy(zeros_v, hist_sh.at[pl.ds(s * RT, RT)])
    bc = pl.multiple_of((c * NS + s) * DCH, DCH)
    pltpu.sync_copy(dstc_hbm.at[pl.ds(bc, DCH)], idx_v)
    plsc.subcore_barrier()
    for b in range(DCH // 8):
        descs = [
            pltpu.async_copy(ones_v, hist_sh.at[idx_v.at[b * 8 + q]], sd,
                             add=True)
            for q in range(8)
        ]
        for d in descs:
            d.wait()
    plsc.subcore_barrier()
    out_off = pl.multiple_of(c * NP + s * RT, RT)
    pltpu.sync_copy(hist_sh.at[pl.ds(s * RT, RT)], out_hbm.at[pl.ds(out_off, RT)])


_MP_DEPTH = 2


@functools.partial(
    pl.kernel,
    out_type=jax.ShapeDtypeStruct((NC * NP, DH), jnp.float32),
    mesh=_MESH,
    scratch_types=(
        [pltpu.VMEM((_MP_DEPTH, 2, CH), jnp.int32),
         pltpu.VMEM((_MP_DEPTH, CH, DH), jnp.float32),
         pltpu.VMEM_SHARED((NP, DH), jnp.float32)]
        + [pltpu.SemaphoreType.DMA] * (3 * _MP_DEPTH)
    ),
)
def _mp_kernel(h_hbm, sd_hbm, acc_hbm, sd_v, rows_v, acc_sh, *sems):
    sg = sems[0:_MP_DEPTH]
    ss = sems[_MP_DEPTH:2 * _MP_DEPTH]
    si = sems[2 * _MP_DEPTH:3 * _MP_DEPTH]
    c = lax.axis_index("c")
    s = lax.axis_index("s")
    r0 = s * RT
    h_off = pl.multiple_of(c * NP + r0, RT)
    pltpu.sync_copy(h_hbm.at[pl.ds(h_off, RT)], acc_sh.at[pl.ds(r0, RT)])
    plsc.subcore_barrier()
    base = c * ECH + s * ET
    ngrp = ET // _MP_DEPTH

    for q in range(_MP_DEPTH):
        pltpu.sync_copy(sd_hbm.at[base + q], sd_v.at[q])
    for q in range(_MP_DEPTH):
        pltpu.async_copy(h_hbm.at[sd_v.at[q].at[0]], rows_v.at[q], sg[q])

    def body(k, carry):
        gnext = pl.multiple_of((k + 1) * _MP_DEPTH, _MP_DEPTH)
        for q in range(_MP_DEPTH):
            pltpu.make_async_copy(h_hbm.at[sd_v.at[q].at[0]], rows_v.at[q],
                                  sg[q]).wait()
            pltpu.async_copy(rows_v.at[q], acc_sh.at[sd_v.at[q].at[1]], ss[q],
                             add=True)
        for q in range(_MP_DEPTH):
            pltpu.make_async_copy(rows_v.at[q], acc_sh.at[sd_v.at[q].at[1]],
                                  ss[q]).wait()
            pltpu.async_copy(sd_hbm.at[base + gnext + q], sd_v.at[q], si[q])
        for q in range(_MP_DEPTH):
            pltpu.make_async_copy(sd_hbm.at[base + gnext + q], sd_v.at[q],
                                  si[q]).wait()
            pltpu.async_copy(h_hbm.at[sd_v.at[q].at[0]], rows_v.at[q], sg[q])
        return carry

    lax.fori_loop(0, ngrp - 1, body, 0)
    for q in range(_MP_DEPTH):
        pltpu.make_async_copy(h_hbm.at[sd_v.at[q].at[0]], rows_v.at[q],
                              sg[q]).wait()
        pltpu.async_copy(rows_v.at[q], acc_sh.at[sd_v.at[q].at[1]], ss[q],
                         add=True)
    for q in range(_MP_DEPTH):
        pltpu.make_async_copy(rows_v.at[q], acc_sh.at[sd_v.at[q].at[1]],
                              ss[q]).wait()
    plsc.subcore_barrier()
    pltpu.sync_copy(acc_sh.at[pl.ds(r0, RT)], acc_hbm.at[pl.ds(h_off, RT)])


_DC_DEPTH = 2


@functools.partial(
    pl.kernel,
    out_type=jax.ShapeDtypeStruct((NC * P, DH), jnp.float32),
    mesh=_MESH,
    scratch_types=(
        [pltpu.VMEM((_DC_DEPTH, 2, CH), jnp.int32),
         pltpu.VMEM((_DC_DEPTH, CH, DH), jnp.float32),
         pltpu.VMEM((_DC_DEPTH, CH, DH), jnp.float32)]
        + [pltpu.SemaphoreType.DMA] * (4 * _DC_DEPTH)
    ),
)
def _dec_kernel(u_hbm, v_hbm, rc_hbm, g_hbm, rc_v, urows_v, vrows_v, *sems):
    sgu = sems[0:_DC_DEPTH]
    sgv = sems[_DC_DEPTH:2 * _DC_DEPTH]
    sw = sems[2 * _DC_DEPTH:3 * _DC_DEPTH]
    si = sems[3 * _DC_DEPTH:4 * _DC_DEPTH]
    c = lax.axis_index("c")
    s = lax.axis_index("s")
    base = c * PCH + s * PT
    out0 = c * P + s * PT * CH
    ngrp = PT // _DC_DEPTH

    def gathers(q):
        pltpu.async_copy(u_hbm.at[rc_v.at[q].at[0]], urows_v.at[q], sgu[q])
        pltpu.async_copy(v_hbm.at[rc_v.at[q].at[1]], vrows_v.at[q], sgv[q])

    def addrows(q):
        def arow(r, cr):
            for j in range(DH // 16):
                sl = pl.ds(j * 16, 16)
                urows_v[q, r, sl] = urows_v[q, r, sl] + vrows_v[q, r, sl]
            return cr
        lax.fori_loop(0, CH, arow, 0)

    def wout(i, q):
        off = pl.multiple_of(out0 + i * CH, CH)
        return g_hbm.at[pl.ds(off, CH)]

    for q in range(_DC_DEPTH):
        pltpu.sync_copy(rc_hbm.at[base + q], rc_v.at[q])
    for q in range(_DC_DEPTH):
        gathers(q)

    def body(k, carry):
        i0 = pl.multiple_of(k * _DC_DEPTH, _DC_DEPTH)
        nxt = i0 + _DC_DEPTH
        for q in range(_DC_DEPTH):
            pltpu.make_async_copy(u_hbm.at[rc_v.at[q].at[0]], urows_v.at[q],
                                  sgu[q]).wait()
            pltpu.make_async_copy(v_hbm.at[rc_v.at[q].at[1]], vrows_v.at[q],
                                  sgv[q]).wait()
            addrows(q)
            pltpu.async_copy(urows_v.at[q], wout(i0 + q, q), sw[q])
        for q in range(_DC_DEPTH):
            pltpu.make_async_copy(urows_v.at[q], wout(i0 + q, q), sw[q]).wait()
            pltpu.async_copy(rc_hbm.at[base + nxt + q], rc_v.at[q], si[q])
        for q in range(_DC_DEPTH):
            pltpu.make_async_copy(rc_hbm.at[base + nxt + q], rc_v.at[q],
                                  si[q]).wait()
            gathers(q)
        return carry

    lax.fori_loop(0, ngrp - 1, body, 0)
    last = pl.multiple_of((ngrp - 1) * _DC_DEPTH, _DC_DEPTH)
    for q in range(_DC_DEPTH):
        pltpu.make_async_copy(u_hbm.at[rc_v.at[q].at[0]], urows_v.at[q],
                              sgu[q]).wait()
        pltpu.make_async_copy(v_hbm.at[rc_v.at[q].at[1]], vrows_v.at[q],
                              sgv[q]).wait()
        addrows(q)
        pltpu.async_copy(urows_v.at[q], wout(last + q, q), sw[q])
    for q in range(_DC_DEPTH):
        pltpu.make_async_copy(urows_v.at[q], wout(last + q, q), sw[q]).wait()


def _tc1_body(x_ref, w_ref, degs_ref, h_ref, dinv_ref):
    deg = degs_ref[0] + degs_ref[1] + 1.0
    dinv = lax.rsqrt(deg)
    h = jnp.dot(x_ref[...], w_ref[...], preferred_element_type=jnp.float32) * dinv
    h_ref[0] = h[:, :DH]
    h_ref[1] = h[:, DH:]
    dinv_ref[...] = dinv


def _tc1(x_pad, w1, degs3):
    return pl.pallas_call(
        _tc1_body,
        grid=(NP // BR,),
        in_specs=[
            pl.BlockSpec((BR, D), lambda i: (i, 0)),
            pl.BlockSpec((D, D), lambda i: (0, 0)),
            pl.BlockSpec((NC, BR, 1), lambda i: (0, i, 0)),
        ],
        out_specs=[
            pl.BlockSpec((NC, BR, DH), lambda i: (0, i, 0)),
            pl.BlockSpec((BR, 1), lambda i: (i, 0)),
        ],
        out_shape=[
            jax.ShapeDtypeStruct((NC, NP, DH), jnp.float32),
            jax.ShapeDtypeStruct((NP, 1), jnp.float32),
        ],
    )(x_pad, w1, degs3)


def _tc2_body(acc_ref, dinv_ref, b_ref, w_ref, out_ref):
    accf = jnp.concatenate([acc_ref[0], acc_ref[1]], axis=-1)
    z = jnp.maximum(accf * dinv_ref[...] + b_ref[...], 0.0)
    h = jnp.dot(z, w_ref[...], preferred_element_type=jnp.float32) * dinv_ref[...]
    out_ref[0] = h[:, :DH]
    out_ref[1] = h[:, DH:]


def _tc2(acc3, dinv, b, w):
    return pl.pallas_call(
        _tc2_body,
        grid=(NP // BR,),
        in_specs=[
            pl.BlockSpec((NC, BR, DH), lambda i: (0, i, 0)),
            pl.BlockSpec((BR, 1), lambda i: (i, 0)),
            pl.BlockSpec((1, D), lambda i: (0, 0)),
            pl.BlockSpec((D, D), lambda i: (0, 0)),
        ],
        out_specs=pl.BlockSpec((NC, BR, DH), lambda i: (0, i, 0)),
        out_shape=jax.ShapeDtypeStruct((NC, NP, DH), jnp.float32),
    )(acc3, dinv, b, w)


def _tc3_body(acc_ref, dinv_ref, b_ref, pw1a_ref, pw1b_ref, pb1_ref, u_ref, v_ref):
    accf = jnp.concatenate([acc_ref[0], acc_ref[1]], axis=-1)
    z = jnp.maximum(accf * dinv_ref[...] + b_ref[...], 0.0)
    u = jnp.dot(z, pw1a_ref[...], preferred_element_type=jnp.float32) + pb1_ref[...]
    v = jnp.dot(z, pw1b_ref[...], preferred_element_type=jnp.float32)
    u_ref[0] = u[:, :DH]
    u_ref[1] = u[:, DH:]
    v_ref[0] = v[:, :DH]
    v_ref[1] = v[:, DH:]


def _tc3(acc3, dinv, b, pw1a, pw1b, pb1):
    return pl.pallas_call(
        _tc3_body,
        grid=(NP // BR,),
        in_specs=[
            pl.BlockSpec((NC, BR, DH), lambda i: (0, i, 0)),
            pl.BlockSpec((BR, 1), lambda i: (i, 0)),
            pl.BlockSpec((1, D), lambda i: (0, 0)),
            pl.BlockSpec((D, D), lambda i: (0, 0)),
            pl.BlockSpec((D, D), lambda i: (0, 0)),
            pl.BlockSpec((1, D), lambda i: (0, 0)),
        ],
        out_specs=[
            pl.BlockSpec((NC, BR, DH), lambda i: (0, i, 0)),
            pl.BlockSpec((NC, BR, DH), lambda i: (0, i, 0)),
        ],
        out_shape=[
            jax.ShapeDtypeStruct((NC, NP, DH), jnp.float32),
            jax.ShapeDtypeStruct((NC, NP, DH), jnp.float32),
        ],
    )(acc3, dinv, b, pw1a, pw1b, pb1)


def _tc4_body(g_ref, pw2_ref, pb2_ref, pw3_ref, pb3_ref, out_ref):
    g = jnp.concatenate([g_ref[0], g_ref[1]], axis=-1)
    h1 = jnp.maximum(g, 0.0)
    h2 = jnp.maximum(
        jnp.dot(h1, pw2_ref[...], preferred_element_type=jnp.float32) + pb2_ref[...],
        0.0)
    o = jnp.dot(h2, pw3_ref[...], preferred_element_type=jnp.float32) + pb3_ref[0, 0]
    out_ref[...] = o.reshape(BP // 128, 128)


def _tc4(g3, pw2, pb2, pw3, pb3):
    return pl.pallas_call(
        _tc4_body,
        grid=(P // BP,),
        in_specs=[
            pl.BlockSpec((NC, BP, DH), lambda i: (0, i, 0)),
            pl.BlockSpec((D, DH), lambda i: (0, 0)),
            pl.BlockSpec((1, DH), lambda i: (0, 0)),
            pl.BlockSpec((DH, 1), lambda i: (0, 0)),
            pl.BlockSpec((1, 1), lambda i: (0, 0)),
        ],
        out_specs=pl.BlockSpec((BP // 128, 128), lambda i: (i, 0)),
        out_shape=jax.ShapeDtypeStruct((P // 128, 128), jnp.float32),
    )(g3, pw2, pb2, pw3, pb3)


def kernel(x, edge_index, edge_label_index, W1, b1, W2, b2,
           PW1, PB1, PW2, PB2, PW3, PB3):
    src = edge_index[0].astype(jnp.int32)
    dst = edge_index[1].astype(jnp.int32)
    padidx = N + (jnp.arange(EP - E, dtype=jnp.int32) % (NP - N))
    srcp = jnp.concatenate([src, padidx])
    dstp = jnp.concatenate([dst, padidx])
    s_c = srcp.reshape(ECH, CH)
    d_c = dstp.reshape(ECH, CH)
    sd = jnp.concatenate([
        jnp.stack([s_c, d_c], axis=1),
        jnp.stack([s_c + NP, d_c], axis=1),
    ], axis=0)

    row = edge_label_index[0].astype(jnp.int32).reshape(PCH, CH)
    col = edge_label_index[1].astype(jnp.int32).reshape(PCH, CH)
    rc = jnp.concatenate([
        jnp.stack([row, col], axis=1),
        jnp.stack([row + NP, col + NP], axis=1),
    ], axis=0)

    x_pad = jnp.pad(x, ((0, NP - N), (0, 0)))

    degs = _deg_kernel(d_c)
    h1f, dinv = _tc1(x_pad, W1, degs.reshape(NC, NP, 1))
    acc1 = _mp_kernel(h1f.reshape(NC * NP, DH), sd)
    h2f = _tc2(acc1.reshape(NC, NP, DH), dinv, b1.reshape(1, D), W2)
    acc2 = _mp_kernel(h2f.reshape(NC * NP, DH), sd)
    uf, vf = _tc3(acc2.reshape(NC, NP, DH), dinv, b2.reshape(1, D),
                  PW1[:D], PW1[D:], PB1.reshape(1, D))
    g = _dec_kernel(uf.reshape(NC * NP, DH), vf.reshape(NC * NP, DH), rc)
    out = _tc4(g.reshape(NC, P, DH), PW2, PB2.reshape(1, DH),
               PW3, PB3.reshape(1, 1))
    return out.reshape(-1)


# speedup vs baseline: 12.3145x; 1.0859x over previous
import functools

import jax
import jax.numpy as jnp
from jax import lax
from jax.experimental import pallas as pl
from jax.experimental.pallas import tpu as pltpu
from jax.experimental.pallas import tpu_sc as plsc

N = 10000
NP = 10240
D = 256
DH = 128
E = 160000
EP = 163840
P = 65536
CH = 128
NC = 2
NS = 16
RT = NP // NS
ECH = EP // CH
DCH = ECH // NC // NS
MCH = 80
MECH = EP // MCH
MET = MECH // NS
DEC_CH = 64
PCH = P // DEC_CH
PT = PCH // NS
BR = 1024
BP = 2048

_MESH = plsc.VectorSubcoreMesh(core_axis_name="c", subcore_axis_name="s",
                               num_cores=NC, num_subcores=NS)


@functools.partial(
    pl.kernel,
    out_type=jax.ShapeDtypeStruct((NC * NP,), jnp.float32),
    mesh=_MESH,
    scratch_types=[
        pltpu.VMEM((DCH, CH), jnp.int32),
        pltpu.VMEM((CH,), jnp.float32),
        pltpu.VMEM((RT,), jnp.float32),
        pltpu.VMEM_SHARED((NP,), jnp.float32),
        pltpu.SemaphoreType.DMA,
    ],
)
def _deg_kernel(dstc_hbm, out_hbm, idx_v, ones_v, zeros_v, hist_sh, sd):
    c = lax.axis_index("c")
    s = lax.axis_index("s")
    for j in range(CH // 16):
        ones_v[pl.ds(j * 16, 16)] = jnp.full((16,), 1.0, jnp.float32)
    for j in range(RT // 16):
        zeros_v[pl.ds(j * 16, 16)] = jnp.zeros((16,), jnp.float32)
    pltpu.sync_copy(zeros_v, hist_sh.at[pl.ds(s * RT, RT)])
    bc = pl.multiple_of((c * NS + s) * DCH, DCH)
    pltpu.sync_copy(dstc_hbm.at[pl.ds(bc, DCH)], idx_v)
    plsc.subcore_barrier()
    for b in range(DCH // 8):
        descs = [
            pltpu.async_copy(ones_v, hist_sh.at[idx_v.at[b * 8 + q]], sd,
                             add=True)
            for q in range(8)
        ]
        for d in descs:
            d.wait()
    plsc.subcore_barrier()
    out_off = pl.multiple_of(c * NP + s * RT, RT)
    pltpu.sync_copy(hist_sh.at[pl.ds(s * RT, RT)], out_hbm.at[pl.ds(out_off, RT)])


_MP_DEPTH = 4


@functools.partial(
    pl.kernel,
    out_type=jax.ShapeDtypeStruct((NC * NP, DH), jnp.float32),
    mesh=_MESH,
    scratch_types=(
        [pltpu.VMEM((_MP_DEPTH, 2, MCH), jnp.int32),
         pltpu.VMEM((_MP_DEPTH, MCH, DH), jnp.float32),
         pltpu.VMEM_SHARED((NP, DH), jnp.float32)]
        + [pltpu.SemaphoreType.DMA] * (3 * _MP_DEPTH)
    ),
)
def _mp_kernel(h_hbm, sd_hbm, acc_hbm, sd_v, rows_v, acc_sh, *sems):
    sg = sems[0:_MP_DEPTH]
    ss = sems[_MP_DEPTH:2 * _MP_DEPTH]
    si = sems[2 * _MP_DEPTH:3 * _MP_DEPTH]
    c = lax.axis_index("c")
    s = lax.axis_index("s")
    r0 = s * RT
    h_off = pl.multiple_of(c * NP + r0, RT)
    pltpu.sync_copy(h_hbm.at[pl.ds(h_off, RT)], acc_sh.at[pl.ds(r0, RT)])
    plsc.subcore_barrier()
    base = c * MECH + s * MET
    ngrp = MET // _MP_DEPTH

    for q in range(_MP_DEPTH):
        pltpu.sync_copy(sd_hbm.at[base + q], sd_v.at[q])
    for q in range(_MP_DEPTH):
        pltpu.async_copy(h_hbm.at[sd_v.at[q].at[0]], rows_v.at[q], sg[q])

    def body(k, carry):
        gnext = pl.multiple_of((k + 1) * _MP_DEPTH, _MP_DEPTH)
        for q in range(_MP_DEPTH):
            pltpu.make_async_copy(h_hbm.at[sd_v.at[q].at[0]], rows_v.at[q],
                                  sg[q]).wait()
            pltpu.async_copy(rows_v.at[q], acc_sh.at[sd_v.at[q].at[1]], ss[q],
                             add=True)
        for q in range(_MP_DEPTH):
            pltpu.make_async_copy(rows_v.at[q], acc_sh.at[sd_v.at[q].at[1]],
                                  ss[q]).wait()
            pltpu.async_copy(sd_hbm.at[base + gnext + q], sd_v.at[q], si[q])
        for q in range(_MP_DEPTH):
            pltpu.make_async_copy(sd_hbm.at[base + gnext + q], sd_v.at[q],
                                  si[q]).wait()
            pltpu.async_copy(h_hbm.at[sd_v.at[q].at[0]], rows_v.at[q], sg[q])
        return carry

    lax.fori_loop(0, ngrp - 1, body, 0)
    for q in range(_MP_DEPTH):
        pltpu.make_async_copy(h_hbm.at[sd_v.at[q].at[0]], rows_v.at[q],
                              sg[q]).wait()
        pltpu.async_copy(rows_v.at[q], acc_sh.at[sd_v.at[q].at[1]], ss[q],
                         add=True)
    for q in range(_MP_DEPTH):
        pltpu.make_async_copy(rows_v.at[q], acc_sh.at[sd_v.at[q].at[1]],
                              ss[q]).wait()
    plsc.subcore_barrier()
    pltpu.sync_copy(acc_sh.at[pl.ds(r0, RT)], acc_hbm.at[pl.ds(h_off, RT)])


_DC_DEPTH = 4


@functools.partial(
    pl.kernel,
    out_type=jax.ShapeDtypeStruct((NC * P, DH), jnp.float32),
    mesh=_MESH,
    scratch_types=(
        [pltpu.VMEM((_DC_DEPTH, 2, DEC_CH), jnp.int32),
         pltpu.VMEM((_DC_DEPTH, DEC_CH, DH), jnp.float32),
         pltpu.VMEM((_DC_DEPTH, DEC_CH, DH), jnp.float32)]
        + [pltpu.SemaphoreType.DMA] * (4 * _DC_DEPTH)
    ),
)
def _dec_kernel(u_hbm, v_hbm, rc_hbm, g_hbm, rc_v, urows_v, vrows_v, *sems):
    sgu = sems[0:_DC_DEPTH]
    sgv = sems[_DC_DEPTH:2 * _DC_DEPTH]
    sw = sems[2 * _DC_DEPTH:3 * _DC_DEPTH]
    si = sems[3 * _DC_DEPTH:4 * _DC_DEPTH]
    c = lax.axis_index("c")
    s = lax.axis_index("s")
    base = c * PCH + s * PT
    out0 = c * P + s * PT * DEC_CH
    ngrp = PT // _DC_DEPTH

    def gathers(q):
        pltpu.async_copy(u_hbm.at[rc_v.at[q].at[0]], urows_v.at[q], sgu[q])
        pltpu.async_copy(v_hbm.at[rc_v.at[q].at[1]], vrows_v.at[q], sgv[q])

    def addrows(q):
        def arow(r, cr):
            for j in range(DH // 16):
                sl = pl.ds(j * 16, 16)
                urows_v[q, r, sl] = urows_v[q, r, sl] + vrows_v[q, r, sl]
            return cr
        lax.fori_loop(0, DEC_CH, arow, 0)

    def wout(i, q):
        off = pl.multiple_of(out0 + i * DEC_CH, DEC_CH)
        return g_hbm.at[pl.ds(off, DEC_CH)]

    for q in range(_DC_DEPTH):
        pltpu.sync_copy(rc_hbm.at[base + q], rc_v.at[q])
    for q in range(_DC_DEPTH):
        gathers(q)

    def body(k, carry):
        i0 = pl.multiple_of(k * _DC_DEPTH, _DC_DEPTH)
        nxt = i0 + _DC_DEPTH
        for q in range(_DC_DEPTH):
            pltpu.make_async_copy(u_hbm.at[rc_v.at[q].at[0]], urows_v.at[q],
                                  sgu[q]).wait()
            pltpu.make_async_copy(v_hbm.at[rc_v.at[q].at[1]], vrows_v.at[q],
                                  sgv[q]).wait()
            addrows(q)
            pltpu.async_copy(urows_v.at[q], wout(i0 + q, q), sw[q])
        for q in range(_DC_DEPTH):
            pltpu.make_async_copy(urows_v.at[q], wout(i0 + q, q), sw[q]).wait()
            pltpu.async_copy(rc_hbm.at[base + nxt + q], rc_v.at[q], si[q])
        for q in range(_DC_DEPTH):
            pltpu.make_async_copy(rc_hbm.at[base + nxt + q], rc_v.at[q],
                                  si[q]).wait()
            gathers(q)
        return carry

    lax.fori_loop(0, ngrp - 1, body, 0)
    last = pl.multiple_of((ngrp - 1) * _DC_DEPTH, _DC_DEPTH)
    for q in range(_DC_DEPTH):
        pltpu.make_async_copy(u_hbm.at[rc_v.at[q].at[0]], urows_v.at[q],
                              sgu[q]).wait()
        pltpu.make_async_copy(v_hbm.at[rc_v.at[q].at[1]], vrows_v.at[q],
                              sgv[q]).wait()
        addrows(q)
        pltpu.async_copy(urows_v.at[q], wout(last + q, q), sw[q])
    for q in range(_DC_DEPTH):
        pltpu.make_async_copy(urows_v.at[q], wout(last + q, q), sw[q]).wait()


def _tc1_body(x_ref, w_ref, degs_ref, h_ref, dinv_ref):
    deg = degs_ref[0] + degs_ref[1] + 1.0
    dinv = lax.rsqrt(deg)
    h = jnp.dot(x_ref[...], w_ref[...], preferred_element_type=jnp.float32) * dinv
    h_ref[0] = h[:, :DH]
    h_ref[1] = h[:, DH:]
    dinv_ref[...] = dinv


def _tc1(x_pad, w1, degs3):
    return pl.pallas_call(
        _tc1_body,
        grid=(NP // BR,),
        in_specs=[
            pl.BlockSpec((BR, D), lambda i: (i, 0)),
            pl.BlockSpec((D, D), lambda i: (0, 0)),
            pl.BlockSpec((NC, BR, 1), lambda i: (0, i, 0)),
        ],
        out_specs=[
            pl.BlockSpec((NC, BR, DH), lambda i: (0, i, 0)),
            pl.BlockSpec((BR, 1), lambda i: (i, 0)),
        ],
        out_shape=[
            jax.ShapeDtypeStruct((NC, NP, DH), jnp.float32),
            jax.ShapeDtypeStruct((NP, 1), jnp.float32),
        ],
    )(x_pad, w1, degs3)


def _tc2_body(acc_ref, dinv_ref, b_ref, w_ref, out_ref):
    accf = jnp.concatenate([acc_ref[0], acc_ref[1]], axis=-1)
    z = jnp.maximum(accf * dinv_ref[...] + b_ref[...], 0.0)
    h = jnp.dot(z, w_ref[...], preferred_element_type=jnp.float32) * dinv_ref[...]
    out_ref[0] = h[:, :DH]
    out_ref[1] = h[:, DH:]


def _tc2(acc3, dinv, b, w):
    return pl.pallas_call(
        _tc2_body,
        grid=(NP // BR,),
        in_specs=[
            pl.BlockSpec((NC, BR, DH), lambda i: (0, i, 0)),
            pl.BlockSpec((BR, 1), lambda i: (i, 0)),
            pl.BlockSpec((1, D), lambda i: (0, 0)),
            pl.BlockSpec((D, D), lambda i: (0, 0)),
        ],
        out_specs=pl.BlockSpec((NC, BR, DH), lambda i: (0, i, 0)),
        out_shape=jax.ShapeDtypeStruct((NC, NP, DH), jnp.float32),
    )(acc3, dinv, b, w)


def _tc3_body(acc_ref, dinv_ref, b_ref, pw1a_ref, pw1b_ref, pb1_ref, u_ref, v_ref):
    accf = jnp.concatenate([acc_ref[0], acc_ref[1]], axis=-1)
    z = jnp.maximum(accf * dinv_ref[...] + b_ref[...], 0.0)
    u = jnp.dot(z, pw1a_ref[...], preferred_element_type=jnp.float32) + pb1_ref[...]
    v = jnp.dot(z, pw1b_ref[...], preferred_element_type=jnp.float32)
    u_ref[0] = u[:, :DH]
    u_ref[1] = u[:, DH:]
    v_ref[0] = v[:, :DH]
    v_ref[1] = v[:, DH:]


def _tc3(acc3, dinv, b, pw1a, pw1b, pb1):
    return pl.pallas_call(
        _tc3_body,
        grid=(NP // BR,),
        in_specs=[
            pl.BlockSpec((NC, BR, DH), lambda i: (0, i, 0)),
            pl.BlockSpec((BR, 1), lambda i: (i, 0)),
            pl.BlockSpec((1, D), lambda i: (0, 0)),
            pl.BlockSpec((D, D), lambda i: (0, 0)),
            pl.BlockSpec((D, D), lambda i: (0, 0)),
            pl.BlockSpec((1, D), lambda i: (0, 0)),
        ],
        out_specs=[
            pl.BlockSpec((NC, BR, DH), lambda i: (0, i, 0)),
            pl.BlockSpec((NC, BR, DH), lambda i: (0, i, 0)),
        ],
        out_shape=[
            jax.ShapeDtypeStruct((NC, NP, DH), jnp.float32),
            jax.ShapeDtypeStruct((NC, NP, DH), jnp.float32),
        ],
    )(acc3, dinv, b, pw1a, pw1b, pb1)


def _tc4_body(g_ref, pw2_ref, pb2_ref, pw3_ref, pb3_ref, out_ref):
    g = jnp.concatenate([g_ref[0], g_ref[1]], axis=-1)
    h1 = jnp.maximum(g, 0.0)
    h2 = jnp.maximum(
        jnp.dot(h1, pw2_ref[...], preferred_element_type=jnp.float32) + pb2_ref[...],
        0.0)
    o = jnp.dot(h2, pw3_ref[...], preferred_element_type=jnp.float32) + pb3_ref[0, 0]
    out_ref[...] = o.reshape(BP // 128, 128)


def _tc4(g3, pw2, pb2, pw3, pb3):
    return pl.pallas_call(
        _tc4_body,
        grid=(P // BP,),
        in_specs=[
            pl.BlockSpec((NC, BP, DH), lambda i: (0, i, 0)),
            pl.BlockSpec((D, DH), lambda i: (0, 0)),
            pl.BlockSpec((1, DH), lambda i: (0, 0)),
            pl.BlockSpec((DH, 1), lambda i: (0, 0)),
            pl.BlockSpec((1, 1), lambda i: (0, 0)),
        ],
        out_specs=pl.BlockSpec((BP // 128, 128), lambda i: (i, 0)),
        out_shape=jax.ShapeDtypeStruct((P // 128, 128), jnp.float32),
    )(g3, pw2, pb2, pw3, pb3)


def kernel(x, edge_index, edge_label_index, W1, b1, W2, b2,
           PW1, PB1, PW2, PB2, PW3, PB3):
    src = edge_index[0].astype(jnp.int32)
    dst = edge_index[1].astype(jnp.int32)
    padidx = N + (jnp.arange(EP - E, dtype=jnp.int32) % (NP - N))
    srcp = jnp.concatenate([src, padidx])
    dstp = jnp.concatenate([dst, padidx])
    d_c = dstp.reshape(ECH, CH)
    s_m = srcp.reshape(MECH, MCH)
    d_m = dstp.reshape(MECH, MCH)
    sd = jnp.concatenate([
        jnp.stack([s_m, d_m], axis=1),
        jnp.stack([s_m + NP, d_m], axis=1),
    ], axis=0)

    row = edge_label_index[0].astype(jnp.int32).reshape(PCH, DEC_CH)
    col = edge_label_index[1].astype(jnp.int32).reshape(PCH, DEC_CH)
    rc = jnp.concatenate([
        jnp.stack([row, col], axis=1),
        jnp.stack([row + NP, col + NP], axis=1),
    ], axis=0)

    x_pad = jnp.pad(x, ((0, NP - N), (0, 0)))

    degs = _deg_kernel(d_c)
    h1f, dinv = _tc1(x_pad, W1, degs.reshape(NC, NP, 1))
    acc1 = _mp_kernel(h1f.reshape(NC * NP, DH), sd)
    h2f = _tc2(acc1.reshape(NC, NP, DH), dinv, b1.reshape(1, D), W2)
    acc2 = _mp_kernel(h2f.reshape(NC * NP, DH), sd)
    uf, vf = _tc3(acc2.reshape(NC, NP, DH), dinv, b2.reshape(1, D),
                  PW1[:D], PW1[D:], PB1.reshape(1, D))
    g = _dec_kernel(uf.reshape(NC * NP, DH), vf.reshape(NC * NP, DH), rc)
    out = _tc4(g.reshape(NC, P, DH), PW2, PB2.reshape(1, DH),
               PW3, PB3.reshape(1, 1))
    return out.reshape(-1)
